# baseline (speedup 1.0000x reference)
import functools

import jax
import jax.numpy as jnp
from jax import lax
from jax.experimental import pallas as pl
from jax.experimental.pallas import tpu as pltpu
from jax.experimental.pallas import tpu_sc as plsc

N = 10000; E = 320000; N2 = 100000; A = 200000; E2 = 800000
F0 = 128; HU = 32; H2 = 64; ISO = 16; G = 256; C = 10

NC = 2
NS = 16


def _round_up(x, m):
    return (x + m - 1) // m * m


def _pelu(x):
    return jnp.where(x > 0, x, jnp.exp(jnp.minimum(x, 0.0)) - 1.0)


@functools.lru_cache(maxsize=None)
def _make_sc_seg_sum(n_chunks, e_pad, n_dst, block_e, w, edge_split, rps):
    assert block_e % rps == 0 and rps <= 128
    R = block_e // rps
    er = e_pad // rps
    n_split = NC * NS if edge_split else NS
    ept = er // n_split
    nb = ept // R
    nb2 = nb // 2
    assert nb2 * 2 * R * n_split == er and nb2 > 0
    n_acc = _round_up(n_dst + 1, 2048)
    rpt = n_acc // NS
    n_tasks = 1 if edge_split else n_chunks

    mesh = plsc.VectorSubcoreMesh(core_axis_name="c", subcore_axis_name="s")

    def body(*refs):
        vcs = refs[:n_chunks]
        src_hbm, dst_hbm, zeros_hbm, out_hbm = refs[n_chunks:n_chunks + 4]
        (src_a, dst_a, rows_a, src_b, dst_b, rows_b, acc,
         gsem_a, gsem_b, ssem_a, ssem_b) = refs[n_chunks + 4:]
        c = lax.axis_index("c")
        s = lax.axis_index("s")

        def idx_copy(sv, dv, base, b):
            row0 = base + b * R
            pltpu.sync_copy(src_hbm.at[pl.ds(row0, R)], sv)
            pltpu.sync_copy(dst_hbm.at[pl.ds(row0, R)], dv)

        def fire_gathers(k, sv, rv, sem):
            for j in range(R):
                pltpu.async_copy(vcs[k].at[sv.at[j]], rv.at[j], sem)

        def fire_scatters(dv, rv, sem):
            for j in range(R):
                pltpu.async_copy(rv.at[j], acc.at[dv.at[j]], sem, add=True)

        def drain(rv, sem):
            for j in range(R):
                pltpu.make_async_copy(zeros_hbm.at[pl.ds(0, rps)],
                                      rv.at[j], sem).wait()

        def run_task(k, edge_base, out_base):
            pltpu.sync_copy(zeros_hbm.at[pl.ds(s * rpt, rpt)],
                            acc.at[pl.ds(s * rpt, rpt)])
            plsc.subcore_barrier()

            idx_copy(src_a, dst_a, edge_base, 0)
            fire_gathers(k, src_a, rows_a, gsem_a)

            def it(i, _):
                b0 = 2 * i
                b1 = 2 * i + 1
                @pl.when(i > 0)
                def _():
                    drain(rows_b, ssem_b)
                idx_copy(src_b, dst_b, edge_base, b1)
                drain(rows_a, gsem_a)
                fire_scatters(dst_a, rows_a, ssem_a)
                fire_gathers(k, src_b, rows_b, gsem_b)
                drain(rows_a, ssem_a)

                @pl.when(i < nb2 - 1)
                def _():
                    idx_copy(src_a, dst_a, edge_base, b0 + 2)
                drain(rows_b, gsem_b)
                fire_scatters(dst_b, rows_b, ssem_b)

                @pl.when(i < nb2 - 1)
                def _():
                    fire_gathers(k, src_a, rows_a, gsem_a)
                return 0

            lax.fori_loop(0, nb2, it, 0)
            drain(rows_b, ssem_b)
            plsc.subcore_barrier()
            pltpu.sync_copy(acc.at[pl.ds(s * rpt, rpt)],
                            out_hbm.at[pl.ds(out_base + s * rpt, rpt)])

        if edge_split:
            run_task(0, c * (er // NC) + s * ept, c * n_acc)
        else:
            for k in range(n_chunks):
                @pl.when(c == (k % NC))
                def _(k=k):
                    run_task(k, s * ept, k * n_acc)

    n_out = (NC if edge_split else n_chunks) * n_acc
    fn = pl.kernel(
        body,
        out_type=jax.ShapeDtypeStruct((n_out, w), jnp.float32),
        mesh=mesh,
        compiler_params=pltpu.CompilerParams(use_tc_tiling_on_sc=False),
        scratch_types=[
            pltpu.VMEM((R, rps), jnp.int32),
            pltpu.VMEM((R, rps), jnp.int32),
            pltpu.VMEM((R, rps, w), jnp.float32),
            pltpu.VMEM((R, rps), jnp.int32),
            pltpu.VMEM((R, rps), jnp.int32),
            pltpu.VMEM((R, rps, w), jnp.float32),
            pltpu.VMEM_SHARED((n_acc, w), jnp.float32),
            pltpu.SemaphoreType.DMA,
            pltpu.SemaphoreType.DMA,
            pltpu.SemaphoreType.DMA,
            pltpu.SemaphoreType.DMA,
        ],
    )
    return fn, n_acc


def _pad_idx(src, dst, e_pad, n_dst, rps):
    e = src.shape[0]
    pe = e_pad - e
    if pe:
        src = jnp.concatenate([src, jnp.zeros((pe,), jnp.int32)])
        dst = jnp.concatenate([dst, jnp.full((pe,), n_dst, jnp.int32)])
    return src.reshape(-1, rps), dst.reshape(-1, rps)


def _sc_seg_sum_chunked(chunks, src, dst, n_dst, block_e, rps):
    e_pad = _round_up(src.shape[0], NS * block_e * 2)
    src2, dst2 = _pad_idx(src, dst, e_pad, n_dst, rps)
    fn, n_acc = _make_sc_seg_sum(len(chunks), e_pad, n_dst, block_e, 16,
                                 False, rps)
    zeros = jnp.zeros((n_acc, 16), jnp.float32)
    out = fn(*chunks, src2, dst2, zeros)
    return [lax.slice(out, (k * n_acc, 0), (k * n_acc + n_dst, 16))
            for k in range(len(chunks))]


def _sc_seg_sum_split(vals, src, dst, n_dst, block_e, rps):
    w = vals.shape[1]
    e_pad = _round_up(src.shape[0], NC * NS * block_e * 2)
    src2, dst2 = _pad_idx(src, dst, e_pad, n_dst, rps)
    fn, n_acc = _make_sc_seg_sum(1, e_pad, n_dst, block_e, w, True, rps)
    zeros = jnp.zeros((n_acc, w), jnp.float32)
    out = fn(vals, src2, dst2, zeros)
    return [lax.slice(out, (k * n_acc, 0), (k * n_acc + n_dst, w))
            for k in range(NC)]


_RB = 2000


def _row_spec(rb, w):
    return pl.BlockSpec((rb, w), lambda i: (i, 0))


def _full_spec(shape):
    return pl.BlockSpec(shape, lambda i: (0, 0))


def _tc_matmul_split(x, w_cat, root_w):
    n, kdim = x.shape
    m = w_cat.shape[1]
    rb = _RB if n % _RB == 0 else n

    def kern(x_ref, w_ref, root_ref, rel_ref):
        y = jnp.dot(x_ref[...], w_ref[...], preferred_element_type=jnp.float32)
        root_ref[...] = y[:, :root_w]
        rel_ref[...] = y[:, root_w:]

    return pl.pallas_call(
        kern,
        grid=(n // rb,),
        in_specs=[_row_spec(rb, kdim), _full_spec((kdim, m))],
        out_specs=[_row_spec(rb, root_w), _row_spec(rb, m - root_w)],
        out_shape=[jax.ShapeDtypeStruct((n, root_w), jnp.float32),
                   jax.ShapeDtypeStruct((n, m - root_w), jnp.float32)],
    )(x, w_cat)


def _tc_elu_matmul_split(root, agg_a, agg_b, b, w_cat, root_w):
    n, win = root.shape
    m = w_cat.shape[1]
    rb = _RB if n % _RB == 0 else n

    def kern(root_ref, aa_ref, ab_ref, b_ref, w_ref, root_o, rel_o):
        h = _pelu(root_ref[...] + aa_ref[...] + ab_ref[...] + b_ref[...])
        y = jnp.dot(h, w_ref[...], preferred_element_type=jnp.float32)
        root_o[...] = y[:, :root_w]
        rel_o[...] = y[:, root_w:]

    return pl.pallas_call(
        kern,
        grid=(n // rb,),
        in_specs=[_row_spec(rb, win)] * 3
        + [_full_spec((1, win)), _full_spec((win, m))],
        out_specs=[_row_spec(rb, root_w), _row_spec(rb, m - root_w)],
        out_shape=[jax.ShapeDtypeStruct((n, root_w), jnp.float32),
                   jax.ShapeDtypeStruct((n, m - root_w), jnp.float32)],
    )(root, agg_a, agg_b, b.reshape(1, -1), w_cat)


def _tc_elu_plus_chunks(root, aggs, b):
    n, win = root.shape
    rb = _RB if n % _RB == 0 else n

    def kern(root_ref, *refs):
        a_refs = refs[:4]
        b_ref = refs[4]
        hp_ref = refs[5]
        ch_refs = refs[6:]
        agg = jnp.concatenate([r[...] for r in a_refs], axis=1)
        h = _pelu(root_ref[...] + agg + b_ref[...])
        ones = (lax.broadcasted_iota(jnp.int32, (h.shape[0], 16), 1)
                == 0).astype(jnp.float32)
        hp_ref[...] = jnp.concatenate([h, ones], axis=1)
        for i, r in enumerate(ch_refs):
            r[...] = h[:, 16 * i: 16 * (i + 1)]

    outs = pl.pallas_call(
        kern,
        grid=(n // rb,),
        in_specs=[_row_spec(rb, win)] + [_row_spec(rb, 16)] * 4
        + [_full_spec((1, win))],
        out_specs=[_row_spec(rb, win + 16)] + [_row_spec(rb, 16)] * 4,
        out_shape=[jax.ShapeDtypeStruct((n, win + 16), jnp.float32)]
        + [jax.ShapeDtypeStruct((n, 16), jnp.float32)] * 4,
    )(root, *aggs, b.reshape(1, -1))
    return outs[0], list(outs[1:])


def _tc_elu_plus(root, aggs, b):
    n, win = root.shape
    rb = _RB if n % _RB == 0 else n

    def kern(root_ref, *refs):
        a_refs = refs[:4]
        b_ref = refs[4]
        hp_ref = refs[5]
        agg = jnp.concatenate([r[...] for r in a_refs], axis=1)
        h = _pelu(root_ref[...] + agg + b_ref[...])
        ones = (lax.broadcasted_iota(jnp.int32, (h.shape[0], 16), 1)
                == 0).astype(jnp.float32)
        hp_ref[...] = jnp.concatenate([h, ones], axis=1)

    return pl.pallas_call(
        kern,
        grid=(n // rb,),
        in_specs=[_row_spec(rb, win)] + [_row_spec(rb, 16)] * 4
        + [_full_spec((1, win))],
        out_specs=_row_spec(rb, win + 16),
        out_shape=jax.ShapeDtypeStruct((n, win + 16), jnp.float32),
    )(root, *aggs, b.reshape(1, -1))


def _tc_elu_matmul_from_chunks(root, aggs, b, w_cat, root_w):
    n, win = root.shape
    m = w_cat.shape[1]
    nch = (m - root_w) // 16
    rb = _RB if n % _RB == 0 else n

    def kern(root_ref, *rest):
        a_refs = rest[:len(aggs)]
        b_ref, w_ref = rest[len(aggs)], rest[len(aggs) + 1]
        root_o = rest[len(aggs) + 2]
        ch_refs = rest[len(aggs) + 3:]
        agg = jnp.concatenate([r[...] for r in a_refs], axis=1)
        h = _pelu(root_ref[...] + agg + b_ref[...])
        y = jnp.dot(h, w_ref[...], preferred_element_type=jnp.float32)
        root_o[...] = y[:, :root_w]
        for i, r in enumerate(ch_refs):
            r[...] = y[:, root_w + 16 * i: root_w + 16 * (i + 1)]

    outs = pl.pallas_call(
        kern,
        grid=(n // rb,),
        in_specs=[_row_spec(rb, win)] + [_row_spec(rb, 16)] * len(aggs)
        + [_full_spec((1, win)), _full_spec((win, m))],
        out_specs=[_row_spec(rb, root_w)] + [_row_spec(rb, 16)] * nch,
        out_shape=[jax.ShapeDtypeStruct((n, root_w), jnp.float32)]
        + [jax.ShapeDtypeStruct((n, 16), jnp.float32)] * nch,
    )(root, *aggs, b.reshape(1, -1), w_cat)
    return outs[0], list(outs[1:])


def _tc_mean_concat_matmul(sums, cnt_chunk, iso, w_cat, root_w):
    n = iso.shape[0]
    m = w_cat.shape[1]
    kdim = 16 * len(sums) + iso.shape[1]
    nch = (m - root_w) // 16
    rb = _RB if n % _RB == 0 else n

    def kern(*refs):
        s_refs = refs[:len(sums)]
        cnt_ref, iso_ref, w_ref = (refs[len(sums)], refs[len(sums) + 1],
                                   refs[len(sums) + 2])
        root_o = refs[len(sums) + 3]
        ch_refs = refs[len(sums) + 4:]
        cnt = jnp.maximum(cnt_ref[...][:, 0:1], 1.0)
        hin = jnp.concatenate([r[...] / cnt for r in s_refs] + [iso_ref[...]],
                              axis=1)
        y = jnp.dot(hin, w_ref[...], preferred_element_type=jnp.float32)
        root_o[...] = y[:, :root_w]
        for i, r in enumerate(ch_refs):
            r[...] = y[:, root_w + 16 * i: root_w + 16 * (i + 1)]

    outs = pl.pallas_call(
        kern,
        grid=(n // rb,),
        in_specs=[_row_spec(rb, 16)] * (len(sums) + 1)
        + [_row_spec(rb, iso.shape[1]), _full_spec((kdim, m))],
        out_specs=[_row_spec(rb, root_w)] + [_row_spec(rb, 16)] * nch,
        out_shape=[jax.ShapeDtypeStruct((n, root_w), jnp.float32)]
        + [jax.ShapeDtypeStruct((n, 16), jnp.float32)] * nch,
    )(*sums, cnt_chunk, iso, w_cat)
    return outs[0], list(outs[1:])


def _tc_head(p1a, p1b, p3a, p3b, Wm1, bm1, Wm2, bm2, Wm3, bm3):
    def kern(p1a_ref, p1b_ref, p3a_ref, p3b_ref, w1, b1r, w2, b2r, w3, b3r,
             out_ref):
        s1 = p1a_ref[...] + p1b_ref[...]
        s3 = p3a_ref[...] + p3b_ref[...]
        x1 = s1[:, :64] / jnp.maximum(s1[:, 64:65], 1.0)
        x2 = s3[:, :64] / jnp.maximum(s3[:, 64:65], 1.0)
        z = jnp.concatenate([x1, x2], axis=1)
        z = _pelu(jnp.dot(z, w1[...], preferred_element_type=jnp.float32) + b1r[...])
        z = _pelu(jnp.dot(z, w2[...], preferred_element_type=jnp.float32) + b2r[...])
        z = jnp.dot(z, w3[...], preferred_element_type=jnp.float32) + b3r[...]
        mx = jnp.max(z, axis=1, keepdims=True)
        lse = jnp.log(jnp.sum(jnp.exp(z - mx), axis=1, keepdims=True)) + mx
        out_ref[...] = z - lse

    return pl.pallas_call(
        kern,
        out_shape=jax.ShapeDtypeStruct((G, C), jnp.float32),
    )(p1a, p1b, p3a, p3b, Wm1, bm1.reshape(1, -1), Wm2, bm2.reshape(1, -1),
      Wm3, bm3.reshape(1, -1))


def _split16(x):
    return [lax.slice(x, (0, 16 * i), (x.shape[0], 16 * (i + 1)))
            for i in range(x.shape[1] // 16)]


def kernel(x, edge_index, batch, assignment_index_2, iso_type_2, edge_index_2,
           batch_2, W1_root, W1_rel, b1, W2_root, W2_rel, b2, W3_root, W3_rel,
           b3, W4_root, W4_rel, b4, W5_root, W5_rel, b5, Wm1, bm1, Wm2, bm2,
           Wm3, bm3):
    src, dst = edge_index[0], edge_index[1]
    src2, dst2 = edge_index_2[0], edge_index_2[1]
    row, col = assignment_index_2[0], assignment_index_2[1]

    ones_n = jnp.zeros((N, 16), jnp.float32).at[:, 0].set(1.0)
    iota_n = jnp.arange(N, dtype=jnp.int32)
    iota_n2 = jnp.arange(N2, dtype=jnp.int32)

    root1, xr1 = _tc_matmul_split(x, jnp.concatenate([W1_root, W1_rel], 1), HU)
    agg1 = _sc_seg_sum_chunked(_split16(xr1), src, dst, N, 2048, 64)
    root2, xr2ch = _tc_elu_matmul_from_chunks(
        root1, agg1, b1, jnp.concatenate([W2_root, W2_rel], 1), H2)
    agg2 = _sc_seg_sum_chunked(xr2ch, src, dst, N, 2048, 64)
    root3, xr3ch = _tc_elu_matmul_from_chunks(
        root2, agg2, b2, jnp.concatenate([W3_root, W3_rel], 1), H2)
    agg3 = _sc_seg_sum_chunked(xr3ch, src, dst, N, 2048, 64)
    hp, hch = _tc_elu_plus_chunks(root3, agg3, b3)

    p1a, p1b = _sc_seg_sum_split(hp, iota_n, batch, G, 512, 32)
    p2 = _sc_seg_sum_chunked(hch + [ones_n], row, col, N2, 768, 32)

    root4, xr4 = _tc_mean_concat_matmul(
        p2[:4], p2[4], iso_type_2, jnp.concatenate([W4_root, W4_rel], 1), H2)
    agg4 = _sc_seg_sum_chunked(xr4, src2, dst2, N2, 768, 32)
    root5, xr5 = _tc_elu_matmul_from_chunks(
        root4, agg4, b4, jnp.concatenate([W5_root, W5_rel], 1), H2)
    agg5 = _sc_seg_sum_chunked(xr5, src2, dst2, N2, 768, 32)
    h2p = _tc_elu_plus(root5, agg5, b5)

    p3a, p3b = _sc_seg_sum_split(h2p, iota_n2, batch_2, G, 512, 32)

    return _tc_head(p1a, p1b, p3a, p3b, Wm1, bm1, Wm2, bm2, Wm3, bm3)


# speedup vs baseline: 1.2308x; 1.2308x over previous
import functools

import jax
import jax.numpy as jnp
from jax import lax
from jax.experimental import pallas as pl
from jax.experimental.pallas import tpu as pltpu
from jax.experimental.pallas import tpu_sc as plsc

N = 10000; E = 320000; N2 = 100000; A = 200000; E2 = 800000
F0 = 128; HU = 32; H2 = 64; ISO = 16; G = 256; C = 10

NC = 2
NS = 16


def _round_up(x, m):
    return (x + m - 1) // m * m


def _pelu(x):
    return jnp.where(x > 0, x, jnp.exp(jnp.minimum(x, 0.0)) - 1.0)


@functools.lru_cache(maxsize=None)
def _make_sc_seg_sum(n_chunks, e_pad, n_dst, block_e, w, edge_split, rps):
    assert block_e % rps == 0 and rps <= 128
    R = block_e // rps
    er = e_pad // rps
    n_split = NC * NS if edge_split else NS
    ept = er // n_split
    nb = ept // R
    nb2 = nb // 2
    assert nb2 * 2 * R * n_split == er and nb2 > 0
    n_acc = _round_up(n_dst + 1, 2048)
    rpt = n_acc // NS
    n_tasks = 1 if edge_split else n_chunks

    mesh = plsc.VectorSubcoreMesh(core_axis_name="c", subcore_axis_name="s")

    def body(*refs):
        vcs = refs[:n_chunks]
        src_hbm, dst_hbm, zeros_hbm, out_hbm = refs[n_chunks:n_chunks + 4]
        (src_a, dst_a, rows_a, src_b, dst_b, rows_b, acc,
         gsem_a, gsem_b, ssem_a, ssem_b) = refs[n_chunks + 4:]
        c = lax.axis_index("c")
        s = lax.axis_index("s")

        def idx_copy(sv, dv, base, b):
            row0 = base + b * R
            pltpu.sync_copy(src_hbm.at[pl.ds(row0, R)], sv)
            pltpu.sync_copy(dst_hbm.at[pl.ds(row0, R)], dv)

        def fire_gathers(k, sv, rv, sem):
            for j in range(R):
                pltpu.async_copy(vcs[k].at[sv.at[j]], rv.at[j], sem)

        def fire_scatters(dv, rv, sem):
            for j in range(R):
                pltpu.async_copy(rv.at[j], acc.at[dv.at[j]], sem, add=True)

        def drain(rv, sem):
            for j in range(R):
                pltpu.make_async_copy(zeros_hbm.at[pl.ds(0, rps)],
                                      rv.at[j], sem).wait()

        def run_task(k, edge_base, out_base):
            pltpu.sync_copy(zeros_hbm.at[pl.ds(s * rpt, rpt)],
                            acc.at[pl.ds(s * rpt, rpt)])
            plsc.subcore_barrier()

            idx_copy(src_a, dst_a, edge_base, 0)
            fire_gathers(k, src_a, rows_a, gsem_a)

            def it(i, _):
                b0 = 2 * i
                b1 = 2 * i + 1
                @pl.when(i > 0)
                def _():
                    drain(rows_b, ssem_b)
                idx_copy(src_b, dst_b, edge_base, b1)
                drain(rows_a, gsem_a)
                fire_scatters(dst_a, rows_a, ssem_a)
                fire_gathers(k, src_b, rows_b, gsem_b)
                drain(rows_a, ssem_a)

                @pl.when(i < nb2 - 1)
                def _():
                    idx_copy(src_a, dst_a, edge_base, b0 + 2)
                drain(rows_b, gsem_b)
                fire_scatters(dst_b, rows_b, ssem_b)

                @pl.when(i < nb2 - 1)
                def _():
                    fire_gathers(k, src_a, rows_a, gsem_a)
                return 0

            lax.fori_loop(0, nb2, it, 0)
            drain(rows_b, ssem_b)
            plsc.subcore_barrier()
            pltpu.sync_copy(acc.at[pl.ds(s * rpt, rpt)],
                            out_hbm.at[pl.ds(out_base + s * rpt, rpt)])

        if edge_split:
            run_task(0, c * (er // NC) + s * ept, c * n_acc)
        else:
            for k in range(n_chunks):
                @pl.when(c == (k % NC))
                def _(k=k):
                    run_task(k, s * ept, k * n_acc)

    n_out = (NC if edge_split else n_chunks) * n_acc
    fn = pl.kernel(
        body,
        out_type=jax.ShapeDtypeStruct((n_out, w), jnp.float32),
        mesh=mesh,
        compiler_params=pltpu.CompilerParams(use_tc_tiling_on_sc=False),
        scratch_types=[
            pltpu.VMEM((R, rps), jnp.int32),
            pltpu.VMEM((R, rps), jnp.int32),
            pltpu.VMEM((R, rps, w), jnp.float32),
            pltpu.VMEM((R, rps), jnp.int32),
            pltpu.VMEM((R, rps), jnp.int32),
            pltpu.VMEM((R, rps, w), jnp.float32),
            pltpu.VMEM_SHARED((n_acc, w), jnp.float32),
            pltpu.SemaphoreType.DMA,
            pltpu.SemaphoreType.DMA,
            pltpu.SemaphoreType.DMA,
            pltpu.SemaphoreType.DMA,
        ],
    )
    return fn, n_acc


def _pad_idx(src, dst, e_pad, n_dst, rps):
    e = src.shape[0]
    pe = e_pad - e
    if pe:
        src = jnp.concatenate([src, jnp.zeros((pe,), jnp.int32)])
        dst = jnp.concatenate([dst, jnp.full((pe,), n_dst, jnp.int32)])
    return src.reshape(-1, rps), dst.reshape(-1, rps)


def _sc_seg_sum_chunked(chunks, src, dst, n_dst, block_e, rps):
    e_pad = _round_up(src.shape[0], NS * block_e * 2)
    src2, dst2 = _pad_idx(src, dst, e_pad, n_dst, rps)
    fn, n_acc = _make_sc_seg_sum(len(chunks), e_pad, n_dst, block_e, 16,
                                 False, rps)
    zeros = jnp.zeros((n_acc, 16), jnp.float32)
    out = fn(*chunks, src2, dst2, zeros)
    return [lax.slice(out, (k * n_acc, 0), (k * n_acc + n_dst, 16))
            for k in range(len(chunks))]


def _sc_seg_sum_split(vals, src, dst, n_dst, block_e, rps):
    w = vals.shape[1]
    e_pad = _round_up(src.shape[0], NC * NS * block_e * 2)
    src2, dst2 = _pad_idx(src, dst, e_pad, n_dst, rps)
    fn, n_acc = _make_sc_seg_sum(1, e_pad, n_dst, block_e, w, True, rps)
    zeros = jnp.zeros((n_acc, w), jnp.float32)
    out = fn(vals, src2, dst2, zeros)
    return [lax.slice(out, (k * n_acc, 0), (k * n_acc + n_dst, w))
            for k in range(NC)]


_RB = 2000


def _row_spec(rb, w):
    return pl.BlockSpec((rb, w), lambda i: (i, 0))


def _full_spec(shape):
    return pl.BlockSpec(shape, lambda i: (0, 0))


def _tc_matmul_split(x, w_cat, root_w):
    n, kdim = x.shape
    m = w_cat.shape[1]
    rb = _RB if n % _RB == 0 else n

    def kern(x_ref, w_ref, root_ref, rel_ref):
        y = jnp.dot(x_ref[...], w_ref[...], preferred_element_type=jnp.float32)
        root_ref[...] = y[:, :root_w]
        rel_ref[...] = y[:, root_w:]

    return pl.pallas_call(
        kern,
        grid=(n // rb,),
        in_specs=[_row_spec(rb, kdim), _full_spec((kdim, m))],
        out_specs=[_row_spec(rb, root_w), _row_spec(rb, m - root_w)],
        out_shape=[jax.ShapeDtypeStruct((n, root_w), jnp.float32),
                   jax.ShapeDtypeStruct((n, m - root_w), jnp.float32)],
    )(x, w_cat)


def _tc_elu_matmul_split(root, agg_a, agg_b, b, w_cat, root_w):
    n, win = root.shape
    m = w_cat.shape[1]
    rb = _RB if n % _RB == 0 else n

    def kern(root_ref, aa_ref, ab_ref, b_ref, w_ref, root_o, rel_o):
        h = _pelu(root_ref[...] + aa_ref[...] + ab_ref[...] + b_ref[...])
        y = jnp.dot(h, w_ref[...], preferred_element_type=jnp.float32)
        root_o[...] = y[:, :root_w]
        rel_o[...] = y[:, root_w:]

    return pl.pallas_call(
        kern,
        grid=(n // rb,),
        in_specs=[_row_spec(rb, win)] * 3
        + [_full_spec((1, win)), _full_spec((win, m))],
        out_specs=[_row_spec(rb, root_w), _row_spec(rb, m - root_w)],
        out_shape=[jax.ShapeDtypeStruct((n, root_w), jnp.float32),
                   jax.ShapeDtypeStruct((n, m - root_w), jnp.float32)],
    )(root, agg_a, agg_b, b.reshape(1, -1), w_cat)


def _tc_elu_plus_chunks(root, aggs, b):
    n, win = root.shape
    rb = _RB if n % _RB == 0 else n

    def kern(root_ref, *refs):
        a_refs = refs[:4]
        b_ref = refs[4]
        hp_ref = refs[5]
        ch_refs = refs[6:]
        agg = jnp.concatenate([r[...] for r in a_refs], axis=1)
        h = _pelu(root_ref[...] + agg + b_ref[...])
        ones = (lax.broadcasted_iota(jnp.int32, (h.shape[0], 16), 1)
                == 0).astype(jnp.float32)
        hp_ref[...] = jnp.concatenate([h, ones], axis=1)
        for i, r in enumerate(ch_refs):
            r[...] = h[:, 16 * i: 16 * (i + 1)]

    outs = pl.pallas_call(
        kern,
        grid=(n // rb,),
        in_specs=[_row_spec(rb, win)] + [_row_spec(rb, 16)] * 4
        + [_full_spec((1, win))],
        out_specs=[_row_spec(rb, win + 16)] + [_row_spec(rb, 16)] * 4,
        out_shape=[jax.ShapeDtypeStruct((n, win + 16), jnp.float32)]
        + [jax.ShapeDtypeStruct((n, 16), jnp.float32)] * 4,
    )(root, *aggs, b.reshape(1, -1))
    return outs[0], list(outs[1:])


def _tc_elu_chunks(root, aggs, b):
    n, win = root.shape
    nch = win // 16
    rb = _RB if n % _RB == 0 else n

    def kern(root_ref, *rest):
        a_refs = rest[:len(aggs)]
        b_ref = rest[len(aggs)]
        ch_refs = rest[len(aggs) + 1:]
        agg = jnp.concatenate([r[...] for r in a_refs], axis=1)
        h = _pelu(root_ref[...] + agg + b_ref[...])
        for i, r in enumerate(ch_refs):
            r[...] = h[:, 16 * i: 16 * (i + 1)]

    outs = pl.pallas_call(
        kern,
        grid=(n // rb,),
        in_specs=[_row_spec(rb, win)] + [_row_spec(rb, 16)] * len(aggs)
        + [_full_spec((1, win))],
        out_specs=[_row_spec(rb, 16)] * nch,
        out_shape=[jax.ShapeDtypeStruct((n, 16), jnp.float32)] * nch,
    )(root, *aggs, b.reshape(1, -1))
    return list(outs)


def _tc_elu_plus(root, aggs, b):
    n, win = root.shape
    rb = _RB if n % _RB == 0 else n

    def kern(root_ref, *refs):
        a_refs = refs[:4]
        b_ref = refs[4]
        hp_ref = refs[5]
        agg = jnp.concatenate([r[...] for r in a_refs], axis=1)
        h = _pelu(root_ref[...] + agg + b_ref[...])
        ones = (lax.broadcasted_iota(jnp.int32, (h.shape[0], 16), 1)
                == 0).astype(jnp.float32)
        hp_ref[...] = jnp.concatenate([h, ones], axis=1)

    return pl.pallas_call(
        kern,
        grid=(n // rb,),
        in_specs=[_row_spec(rb, win)] + [_row_spec(rb, 16)] * 4
        + [_full_spec((1, win))],
        out_specs=_row_spec(rb, win + 16),
        out_shape=jax.ShapeDtypeStruct((n, win + 16), jnp.float32),
    )(root, *aggs, b.reshape(1, -1))


def _tc_elu_matmul_from_chunks(root, aggs, b, w_cat, root_w):
    n, win = root.shape
    m = w_cat.shape[1]
    nch = (m - root_w) // 16
    rb = _RB if n % _RB == 0 else n

    def kern(root_ref, *rest):
        a_refs = rest[:len(aggs)]
        b_ref, w_ref = rest[len(aggs)], rest[len(aggs) + 1]
        root_o = rest[len(aggs) + 2]
        ch_refs = rest[len(aggs) + 3:]
        agg = jnp.concatenate([r[...] for r in a_refs], axis=1)
        h = _pelu(root_ref[...] + agg + b_ref[...])
        y = jnp.dot(h, w_ref[...], preferred_element_type=jnp.float32)
        root_o[...] = y[:, :root_w]
        for i, r in enumerate(ch_refs):
            r[...] = y[:, root_w + 16 * i: root_w + 16 * (i + 1)]

    outs = pl.pallas_call(
        kern,
        grid=(n // rb,),
        in_specs=[_row_spec(rb, win)] + [_row_spec(rb, 16)] * len(aggs)
        + [_full_spec((1, win)), _full_spec((win, m))],
        out_specs=[_row_spec(rb, root_w)] + [_row_spec(rb, 16)] * nch,
        out_shape=[jax.ShapeDtypeStruct((n, root_w), jnp.float32)]
        + [jax.ShapeDtypeStruct((n, 16), jnp.float32)] * nch,
    )(root, *aggs, b.reshape(1, -1), w_cat)
    return outs[0], list(outs[1:])


def _tc_mean_concat_matmul(sums, cnt_chunk, iso, w_cat, root_w):
    n = iso.shape[0]
    m = w_cat.shape[1]
    kdim = 16 * len(sums) + iso.shape[1]
    nch = (m - root_w) // 16
    rb = _RB if n % _RB == 0 else n

    def kern(*refs):
        s_refs = refs[:len(sums)]
        cnt_ref, iso_ref, w_ref = (refs[len(sums)], refs[len(sums) + 1],
                                   refs[len(sums) + 2])
        root_o = refs[len(sums) + 3]
        ch_refs = refs[len(sums) + 4:]
        cnt = jnp.maximum(cnt_ref[...][:, 0:1], 1.0)
        hin = jnp.concatenate([r[...] / cnt for r in s_refs] + [iso_ref[...]],
                              axis=1)
        y = jnp.dot(hin, w_ref[...], preferred_element_type=jnp.float32)
        root_o[...] = y[:, :root_w]
        for i, r in enumerate(ch_refs):
            r[...] = y[:, root_w + 16 * i: root_w + 16 * (i + 1)]

    outs = pl.pallas_call(
        kern,
        grid=(n // rb,),
        in_specs=[_row_spec(rb, 16)] * (len(sums) + 1)
        + [_row_spec(rb, iso.shape[1]), _full_spec((kdim, m))],
        out_specs=[_row_spec(rb, root_w)] + [_row_spec(rb, 16)] * nch,
        out_shape=[jax.ShapeDtypeStruct((n, root_w), jnp.float32)]
        + [jax.ShapeDtypeStruct((n, 16), jnp.float32)] * nch,
    )(*sums, cnt_chunk, iso, w_cat)
    return outs[0], list(outs[1:])


def _tc_head(s1, c1, s2, c2, Wm1, bm1, Wm2, bm2, Wm3, bm3):
    def kern(*refs):
        s1_refs = refs[0:4]
        c1_ref = refs[4]
        s2_refs = refs[5:9]
        c2_ref = refs[9]
        w1, b1r, w2, b2r, w3, b3r, out_ref = refs[10:]
        cnt1 = jnp.maximum(c1_ref[...][:, 0:1], 1.0)
        cnt2 = jnp.maximum(c2_ref[...][:, 0:1], 1.0)
        z = jnp.concatenate([r[...] / cnt1 for r in s1_refs]
                            + [r[...] / cnt2 for r in s2_refs], axis=1)
        z = _pelu(jnp.dot(z, w1[...], preferred_element_type=jnp.float32) + b1r[...])
        z = _pelu(jnp.dot(z, w2[...], preferred_element_type=jnp.float32) + b2r[...])
        z = jnp.dot(z, w3[...], preferred_element_type=jnp.float32) + b3r[...]
        mx = jnp.max(z, axis=1, keepdims=True)
        lse = jnp.log(jnp.sum(jnp.exp(z - mx), axis=1, keepdims=True)) + mx
        out_ref[...] = z - lse

    return pl.pallas_call(
        kern,
        out_shape=jax.ShapeDtypeStruct((G, C), jnp.float32),
    )(*s1, c1, *s2, c2, Wm1, bm1.reshape(1, -1), Wm2, bm2.reshape(1, -1),
      Wm3, bm3.reshape(1, -1))


def _split16(x):
    return [lax.slice(x, (0, 16 * i), (x.shape[0], 16 * (i + 1)))
            for i in range(x.shape[1] // 16)]


def kernel(x, edge_index, batch, assignment_index_2, iso_type_2, edge_index_2,
           batch_2, W1_root, W1_rel, b1, W2_root, W2_rel, b2, W3_root, W3_rel,
           b3, W4_root, W4_rel, b4, W5_root, W5_rel, b5, Wm1, bm1, Wm2, bm2,
           Wm3, bm3):
    src, dst = edge_index[0], edge_index[1]
    src2, dst2 = edge_index_2[0], edge_index_2[1]
    row, col = assignment_index_2[0], assignment_index_2[1]

    ones_n = jnp.zeros((N, 16), jnp.float32).at[:, 0].set(1.0)
    ones_n2 = jnp.zeros((N2, 16), jnp.float32).at[:, 0].set(1.0)
    iota_n = jnp.arange(N, dtype=jnp.int32)
    iota_n2 = jnp.arange(N2, dtype=jnp.int32)

    root1, xr1 = _tc_matmul_split(x, jnp.concatenate([W1_root, W1_rel], 1), HU)
    agg1 = _sc_seg_sum_chunked(_split16(xr1), src, dst, N, 2048, 128)
    root2, xr2ch = _tc_elu_matmul_from_chunks(
        root1, agg1, b1, jnp.concatenate([W2_root, W2_rel], 1), H2)
    agg2 = _sc_seg_sum_chunked(xr2ch, src, dst, N, 2048, 128)
    root3, xr3ch = _tc_elu_matmul_from_chunks(
        root2, agg2, b2, jnp.concatenate([W3_root, W3_rel], 1), H2)
    agg3 = _sc_seg_sum_chunked(xr3ch, src, dst, N, 2048, 128)
    hch = _tc_elu_chunks(root3, agg3, b3)

    p1 = _sc_seg_sum_chunked(hch + [ones_n], iota_n, batch, G, 512, 128)
    p2 = _sc_seg_sum_chunked(hch + [ones_n], row, col, N2, 768, 128)

    root4, xr4 = _tc_mean_concat_matmul(
        p2[:4], p2[4], iso_type_2, jnp.concatenate([W4_root, W4_rel], 1), H2)
    agg4 = _sc_seg_sum_chunked(xr4, src2, dst2, N2, 768, 128)
    root5, xr5 = _tc_elu_matmul_from_chunks(
        root4, agg4, b4, jnp.concatenate([W5_root, W5_rel], 1), H2)
    agg5 = _sc_seg_sum_chunked(xr5, src2, dst2, N2, 768, 128)
    h2ch = _tc_elu_chunks(root5, agg5, b5)

    p3 = _sc_seg_sum_chunked(h2ch + [ones_n2], iota_n2, batch_2, G, 512, 128)

    return _tc_head(p1[:4], p1[4], p3[:4], p3[4],
                    Wm1, bm1, Wm2, bm2, Wm3, bm3)


# speedup vs baseline: 1.2999x; 1.0561x over previous
import functools

import jax
import jax.numpy as jnp
from jax import lax
from jax.experimental import pallas as pl
from jax.experimental.pallas import tpu as pltpu
from jax.experimental.pallas import tpu_sc as plsc

N = 10000; E = 320000; N2 = 100000; A = 200000; E2 = 800000
F0 = 128; HU = 32; H2 = 64; ISO = 16; G = 256; C = 10

NC = 2
NS = 16


def _round_up(x, m):
    return (x + m - 1) // m * m


def _pelu(x):
    return jnp.where(x > 0, x, jnp.exp(jnp.minimum(x, 0.0)) - 1.0)


@functools.lru_cache(maxsize=None)
def _make_sc_seg_sum(n_chunks, e_pad, n_dst, block_e, w, edge_split, rps):
    assert block_e % rps == 0 and rps <= 128
    R = block_e // rps
    er = e_pad // rps
    n_split = NC * NS if edge_split else NS
    ept = er // n_split
    nb = ept // R
    nb2 = nb // 2
    assert nb2 * 2 * R * n_split == er and nb2 > 0
    n_acc = _round_up(n_dst + 1, 2048)
    rpt = n_acc // NS
    n_tasks = 1 if edge_split else n_chunks

    mesh = plsc.VectorSubcoreMesh(core_axis_name="c", subcore_axis_name="s")

    def body(*refs):
        vcs = refs[:n_chunks]
        src_hbm, dst_hbm, zeros_hbm, out_hbm = refs[n_chunks:n_chunks + 4]
        (src_a, dst_a, rows_a, src_b, dst_b, rows_b, acc,
         gsem_a, gsem_b, ssem_a, ssem_b) = refs[n_chunks + 4:]
        c = lax.axis_index("c")
        s = lax.axis_index("s")

        def idx_copy(sv, dv, base, b):
            row0 = base + b * R
            pltpu.sync_copy(src_hbm.at[pl.ds(row0, R)], sv)
            pltpu.sync_copy(dst_hbm.at[pl.ds(row0, R)], dv)

        def fire_gathers(k, sv, rv, sem):
            for j in range(R):
                pltpu.async_copy(vcs[k].at[sv.at[j]], rv.at[j], sem)

        def fire_scatters(dv, rv, sem):
            for j in range(R):
                pltpu.async_copy(rv.at[j], acc.at[dv.at[j]], sem, add=True)

        def drain(rv, sem):
            for j in range(R):
                pltpu.make_async_copy(zeros_hbm.at[pl.ds(0, rps)],
                                      rv.at[j], sem).wait()

        def run_task(k, edge_base, out_base):
            pltpu.sync_copy(zeros_hbm.at[pl.ds(s * rpt, rpt)],
                            acc.at[pl.ds(s * rpt, rpt)])
            plsc.subcore_barrier()

            idx_copy(src_a, dst_a, edge_base, 0)
            fire_gathers(k, src_a, rows_a, gsem_a)

            def it(i, _):
                b0 = 2 * i
                b1 = 2 * i + 1
                @pl.when(i > 0)
                def _():
                    drain(rows_b, ssem_b)
                idx_copy(src_b, dst_b, edge_base, b1)
                drain(rows_a, gsem_a)
                fire_scatters(dst_a, rows_a, ssem_a)
                fire_gathers(k, src_b, rows_b, gsem_b)
                drain(rows_a, ssem_a)

                @pl.when(i < nb2 - 1)
                def _():
                    idx_copy(src_a, dst_a, edge_base, b0 + 2)
                drain(rows_b, gsem_b)
                fire_scatters(dst_b, rows_b, ssem_b)

                @pl.when(i < nb2 - 1)
                def _():
                    fire_gathers(k, src_a, rows_a, gsem_a)
                return 0

            lax.fori_loop(0, nb2, it, 0)
            drain(rows_b, ssem_b)
            plsc.subcore_barrier()
            pltpu.sync_copy(acc.at[pl.ds(s * rpt, rpt)],
                            out_hbm.at[pl.ds(out_base + s * rpt, rpt)])

        if edge_split:
            run_task(0, c * (er // NC) + s * ept, c * n_acc)
        else:
            for k in range(n_chunks):
                @pl.when(c == (k % NC))
                def _(k=k):
                    run_task(k, s * ept, k * n_acc)

    n_out = (NC if edge_split else n_chunks) * n_acc
    fn = pl.kernel(
        body,
        out_type=jax.ShapeDtypeStruct((n_out, w), jnp.float32),
        mesh=mesh,
        compiler_params=pltpu.CompilerParams(use_tc_tiling_on_sc=False),
        scratch_types=[
            pltpu.VMEM((R, rps), jnp.int32),
            pltpu.VMEM((R, rps), jnp.int32),
            pltpu.VMEM((R, rps, w), jnp.float32),
            pltpu.VMEM((R, rps), jnp.int32),
            pltpu.VMEM((R, rps), jnp.int32),
            pltpu.VMEM((R, rps, w), jnp.float32),
            pltpu.VMEM_SHARED((n_acc, w), jnp.float32),
            pltpu.SemaphoreType.DMA,
            pltpu.SemaphoreType.DMA,
            pltpu.SemaphoreType.DMA,
            pltpu.SemaphoreType.DMA,
        ],
    )
    return fn, n_acc


def _pad_idx(src, dst, e_pad, n_dst, rps):
    e = src.shape[0]
    pe = e_pad - e
    if pe:
        src = jnp.concatenate([src, jnp.zeros((pe,), jnp.int32)])
        dst = jnp.concatenate([dst, jnp.full((pe,), n_dst, jnp.int32)])
    return src.reshape(-1, rps), dst.reshape(-1, rps)


def _sc_seg_sum_chunked(chunks, src, dst, n_dst, block_e, rps):
    e_pad = _round_up(src.shape[0], NS * block_e * 2)
    src2, dst2 = _pad_idx(src, dst, e_pad, n_dst, rps)
    fn, n_acc = _make_sc_seg_sum(len(chunks), e_pad, n_dst, block_e, 16,
                                 False, rps)
    zeros = jnp.zeros((n_acc, 16), jnp.float32)
    out = fn(*chunks, src2, dst2, zeros)
    return [lax.slice(out, (k * n_acc, 0), (k * n_acc + n_dst, 16))
            for k in range(len(chunks))]


def _sc_seg_sum_split(vals, src, dst, n_dst, block_e, rps):
    w = vals.shape[1]
    e_pad = _round_up(src.shape[0], NC * NS * block_e * 2)
    src2, dst2 = _pad_idx(src, dst, e_pad, n_dst, rps)
    fn, n_acc = _make_sc_seg_sum(1, e_pad, n_dst, block_e, w, True, rps)
    zeros = jnp.zeros((n_acc, w), jnp.float32)
    out = fn(vals, src2, dst2, zeros)
    return [lax.slice(out, (k * n_acc, 0), (k * n_acc + n_dst, w))
            for k in range(NC)]


_RB = 2000


def _row_spec(rb, w):
    return pl.BlockSpec((rb, w), lambda i: (i, 0))


def _full_spec(shape):
    return pl.BlockSpec(shape, lambda i: (0, 0))


def _tc_matmul_split(x, w_cat, root_w):
    n, kdim = x.shape
    m = w_cat.shape[1]
    rb = _RB if n % _RB == 0 else n

    def kern(x_ref, w_ref, root_ref, rel_ref):
        y = jnp.dot(x_ref[...], w_ref[...], preferred_element_type=jnp.float32)
        root_ref[...] = y[:, :root_w]
        rel_ref[...] = y[:, root_w:]

    return pl.pallas_call(
        kern,
        grid=(n // rb,),
        in_specs=[_row_spec(rb, kdim), _full_spec((kdim, m))],
        out_specs=[_row_spec(rb, root_w), _row_spec(rb, m - root_w)],
        out_shape=[jax.ShapeDtypeStruct((n, root_w), jnp.float32),
                   jax.ShapeDtypeStruct((n, m - root_w), jnp.float32)],
    )(x, w_cat)


def _tc_elu_matmul_split(root, agg_a, agg_b, b, w_cat, root_w):
    n, win = root.shape
    m = w_cat.shape[1]
    rb = _RB if n % _RB == 0 else n

    def kern(root_ref, aa_ref, ab_ref, b_ref, w_ref, root_o, rel_o):
        h = _pelu(root_ref[...] + aa_ref[...] + ab_ref[...] + b_ref[...])
        y = jnp.dot(h, w_ref[...], preferred_element_type=jnp.float32)
        root_o[...] = y[:, :root_w]
        rel_o[...] = y[:, root_w:]

    return pl.pallas_call(
        kern,
        grid=(n // rb,),
        in_specs=[_row_spec(rb, win)] * 3
        + [_full_spec((1, win)), _full_spec((win, m))],
        out_specs=[_row_spec(rb, root_w), _row_spec(rb, m - root_w)],
        out_shape=[jax.ShapeDtypeStruct((n, root_w), jnp.float32),
                   jax.ShapeDtypeStruct((n, m - root_w), jnp.float32)],
    )(root, agg_a, agg_b, b.reshape(1, -1), w_cat)


def _tc_elu_plus_chunks(root, aggs, b):
    n, win = root.shape
    rb = _RB if n % _RB == 0 else n

    def kern(root_ref, *refs):
        a_refs = refs[:4]
        b_ref = refs[4]
        hp_ref = refs[5]
        ch_refs = refs[6:]
        agg = jnp.concatenate([r[...] for r in a_refs], axis=1)
        h = _pelu(root_ref[...] + agg + b_ref[...])
        ones = (lax.broadcasted_iota(jnp.int32, (h.shape[0], 16), 1)
                == 0).astype(jnp.float32)
        hp_ref[...] = jnp.concatenate([h, ones], axis=1)
        for i, r in enumerate(ch_refs):
            r[...] = h[:, 16 * i: 16 * (i + 1)]

    outs = pl.pallas_call(
        kern,
        grid=(n // rb,),
        in_specs=[_row_spec(rb, win)] + [_row_spec(rb, 16)] * 4
        + [_full_spec((1, win))],
        out_specs=[_row_spec(rb, win + 16)] + [_row_spec(rb, 16)] * 4,
        out_shape=[jax.ShapeDtypeStruct((n, win + 16), jnp.float32)]
        + [jax.ShapeDtypeStruct((n, 16), jnp.float32)] * 4,
    )(root, *aggs, b.reshape(1, -1))
    return outs[0], list(outs[1:])


def _tc_elu_chunks(root, aggs, b):
    n, win = root.shape
    nch = win // 16
    rb = _RB if n % _RB == 0 else n

    def kern(root_ref, *rest):
        a_refs = rest[:len(aggs)]
        b_ref = rest[len(aggs)]
        ch_refs = rest[len(aggs) + 1:]
        agg = jnp.concatenate([r[...] for r in a_refs], axis=1)
        h = _pelu(root_ref[...] + agg + b_ref[...])
        for i, r in enumerate(ch_refs):
            r[...] = h[:, 16 * i: 16 * (i + 1)]

    outs = pl.pallas_call(
        kern,
        grid=(n // rb,),
        in_specs=[_row_spec(rb, win)] + [_row_spec(rb, 16)] * len(aggs)
        + [_full_spec((1, win))],
        out_specs=[_row_spec(rb, 16)] * nch,
        out_shape=[jax.ShapeDtypeStruct((n, 16), jnp.float32)] * nch,
    )(root, *aggs, b.reshape(1, -1))
    return list(outs)


def _tc_elu_plus(root, aggs, b):
    n, win = root.shape
    rb = _RB if n % _RB == 0 else n

    def kern(root_ref, *refs):
        a_refs = refs[:4]
        b_ref = refs[4]
        hp_ref = refs[5]
        agg = jnp.concatenate([r[...] for r in a_refs], axis=1)
        h = _pelu(root_ref[...] + agg + b_ref[...])
        ones = (lax.broadcasted_iota(jnp.int32, (h.shape[0], 16), 1)
                == 0).astype(jnp.float32)
        hp_ref[...] = jnp.concatenate([h, ones], axis=1)

    return pl.pallas_call(
        kern,
        grid=(n // rb,),
        in_specs=[_row_spec(rb, win)] + [_row_spec(rb, 16)] * 4
        + [_full_spec((1, win))],
        out_specs=_row_spec(rb, win + 16),
        out_shape=jax.ShapeDtypeStruct((n, win + 16), jnp.float32),
    )(root, *aggs, b.reshape(1, -1))


def _tc_elu_matmul_from_chunks(root, aggs, b, w_cat, root_w):
    n, win = root.shape
    m = w_cat.shape[1]
    nch = (m - root_w) // 16
    rb = _RB if n % _RB == 0 else n

    def kern(root_ref, *rest):
        a_refs = rest[:len(aggs)]
        b_ref, w_ref = rest[len(aggs)], rest[len(aggs) + 1]
        root_o = rest[len(aggs) + 2]
        ch_refs = rest[len(aggs) + 3:]
        agg = jnp.concatenate([r[...] for r in a_refs], axis=1)
        h = _pelu(root_ref[...] + agg + b_ref[...])
        y = jnp.dot(h, w_ref[...], preferred_element_type=jnp.float32)
        root_o[...] = y[:, :root_w]
        for i, r in enumerate(ch_refs):
            r[...] = y[:, root_w + 16 * i: root_w + 16 * (i + 1)]

    outs = pl.pallas_call(
        kern,
        grid=(n // rb,),
        in_specs=[_row_spec(rb, win)] + [_row_spec(rb, 16)] * len(aggs)
        + [_full_spec((1, win)), _full_spec((win, m))],
        out_specs=[_row_spec(rb, root_w)] + [_row_spec(rb, 16)] * nch,
        out_shape=[jax.ShapeDtypeStruct((n, root_w), jnp.float32)]
        + [jax.ShapeDtypeStruct((n, 16), jnp.float32)] * nch,
    )(root, *aggs, b.reshape(1, -1), w_cat)
    return outs[0], list(outs[1:])


def _tc_mean_concat_matmul(sums, cnt_chunk, iso, w_cat, root_w):
    n = iso.shape[0]
    m = w_cat.shape[1]
    kdim = 16 * len(sums) + iso.shape[1]
    nch = (m - root_w) // 16
    rb = _RB if n % _RB == 0 else n

    def kern(*refs):
        s_refs = refs[:len(sums)]
        cnt_ref, iso_ref, w_ref = (refs[len(sums)], refs[len(sums) + 1],
                                   refs[len(sums) + 2])
        root_o = refs[len(sums) + 3]
        ch_refs = refs[len(sums) + 4:]
        cnt = jnp.maximum(cnt_ref[...][:, 0:1], 1.0)
        hin = jnp.concatenate([r[...] / cnt for r in s_refs] + [iso_ref[...]],
                              axis=1)
        y = jnp.dot(hin, w_ref[...], preferred_element_type=jnp.float32)
        root_o[...] = y[:, :root_w]
        for i, r in enumerate(ch_refs):
            r[...] = y[:, root_w + 16 * i: root_w + 16 * (i + 1)]

    outs = pl.pallas_call(
        kern,
        grid=(n // rb,),
        in_specs=[_row_spec(rb, 16)] * (len(sums) + 1)
        + [_row_spec(rb, iso.shape[1]), _full_spec((kdim, m))],
        out_specs=[_row_spec(rb, root_w)] + [_row_spec(rb, 16)] * nch,
        out_shape=[jax.ShapeDtypeStruct((n, root_w), jnp.float32)]
        + [jax.ShapeDtypeStruct((n, 16), jnp.float32)] * nch,
    )(*sums, cnt_chunk, iso, w_cat)
    return outs[0], list(outs[1:])


def _tc_head(s1, c1, s2, c2, Wm1, bm1, Wm2, bm2, Wm3, bm3):
    def kern(*refs):
        s1_refs = refs[0:4]
        c1_ref = refs[4]
        s2_refs = refs[5:9]
        c2_ref = refs[9]
        w1, b1r, w2, b2r, w3, b3r, out_ref = refs[10:]
        cnt1 = jnp.maximum(c1_ref[...][:, 0:1], 1.0)
        cnt2 = jnp.maximum(c2_ref[...][:, 0:1], 1.0)
        z = jnp.concatenate([r[...] / cnt1 for r in s1_refs]
                            + [r[...] / cnt2 for r in s2_refs], axis=1)
        z = _pelu(jnp.dot(z, w1[...], preferred_element_type=jnp.float32) + b1r[...])
        z = _pelu(jnp.dot(z, w2[...], preferred_element_type=jnp.float32) + b2r[...])
        z = jnp.dot(z, w3[...], preferred_element_type=jnp.float32) + b3r[...]
        mx = jnp.max(z, axis=1, keepdims=True)
        lse = jnp.log(jnp.sum(jnp.exp(z - mx), axis=1, keepdims=True)) + mx
        out_ref[...] = z - lse

    return pl.pallas_call(
        kern,
        out_shape=jax.ShapeDtypeStruct((G, C), jnp.float32),
    )(*s1, c1, *s2, c2, Wm1, bm1.reshape(1, -1), Wm2, bm2.reshape(1, -1),
      Wm3, bm3.reshape(1, -1))


def _split16(x):
    return [lax.slice(x, (0, 16 * i), (x.shape[0], 16 * (i + 1)))
            for i in range(x.shape[1] // 16)]


def kernel(x, edge_index, batch, assignment_index_2, iso_type_2, edge_index_2,
           batch_2, W1_root, W1_rel, b1, W2_root, W2_rel, b2, W3_root, W3_rel,
           b3, W4_root, W4_rel, b4, W5_root, W5_rel, b5, Wm1, bm1, Wm2, bm2,
           Wm3, bm3):
    src, dst = edge_index[0], edge_index[1]
    src2, dst2 = edge_index_2[0], edge_index_2[1]
    row, col = assignment_index_2[0], assignment_index_2[1]

    ones_n = jnp.zeros((N, 16), jnp.float32).at[:, 0].set(1.0)
    ones_n2 = jnp.zeros((N2, 16), jnp.float32).at[:, 0].set(1.0)
    iota_n = jnp.arange(N, dtype=jnp.int32)
    iota_n2 = jnp.arange(N2, dtype=jnp.int32)

    root1, xr1 = _tc_matmul_split(x, jnp.concatenate([W1_root, W1_rel], 1), HU)
    agg1 = _sc_seg_sum_chunked(_split16(xr1), src, dst, N, 2048, 128)
    root2, xr2ch = _tc_elu_matmul_from_chunks(
        root1, agg1, b1, jnp.concatenate([W2_root, W2_rel], 1), H2)
    agg2 = _sc_seg_sum_chunked(xr2ch, src, dst, N, 2048, 128)
    root3, xr3ch = _tc_elu_matmul_from_chunks(
        root2, agg2, b2, jnp.concatenate([W3_root, W3_rel], 1), H2)
    agg3 = _sc_seg_sum_chunked(xr3ch, src, dst, N, 2048, 128)
    hch = _tc_elu_chunks(root3, agg3, b3)

    p1 = _sc_seg_sum_chunked(hch + [ones_n], iota_n, batch, G, 512, 128)
    p2 = _sc_seg_sum_chunked(hch + [ones_n], row, col, N2, 512, 128)

    root4, xr4 = _tc_mean_concat_matmul(
        p2[:4], p2[4], iso_type_2, jnp.concatenate([W4_root, W4_rel], 1), H2)
    agg4 = _sc_seg_sum_chunked(xr4, src2, dst2, N2, 512, 128)
    root5, xr5 = _tc_elu_matmul_from_chunks(
        root4, agg4, b4, jnp.concatenate([W5_root, W5_rel], 1), H2)
    agg5 = _sc_seg_sum_chunked(xr5, src2, dst2, N2, 512, 128)
    h2ch = _tc_elu_chunks(root5, agg5, b5)

    p3 = _sc_seg_sum_chunked(h2ch + [ones_n2], iota_n2, batch_2, G, 512, 128)

    return _tc_head(p1[:4], p1[4], p3[:4], p3[4],
                    Wm1, bm1, Wm2, bm2, Wm3, bm3)
